# baseline (speedup 1.0000x reference)
import functools
import jax
import jax.numpy as jnp
from jax import lax
from jax.experimental import pallas as pl
from jax.experimental.pallas import tpu as pltpu
from jax.experimental.pallas import tpu_sc as plsc

_N_UIDS = 8192
_TOPK = 20
_BATCH = 32
_ROWS = 32 * 128
_D = 512
_UID_BLK = 1024
_ROW_BLK = 128

_NJ = 0
_R_TC = _ROWS - _NJ
_NTILES = 32
_CH = 8
_NRT = _NJ // _NTILES
_USLC = _N_UIDS // _NTILES

_NEG = float("-inf")
_BIGI = 2 ** 30


def _gate_body(qt_ref, w_ref, b_ref, o_ref):
    s = jax.lax.dot_general(w_ref[...], qt_ref[...],
                            (((1,), (0,)), ((), ())),
                            preferred_element_type=jnp.float32)
    mv = jnp.sum(s, axis=1) * (1.0 / _BATCH)
    o_ref[...] = mv[None, None, :] + b_ref[...]


def _gate_stage(query, gate_W, gate_b):
    nblk = _N_UIDS // _UID_BLK
    b3 = gate_b.reshape(nblk, 1, _UID_BLK)
    qt = jnp.swapaxes(query, 0, 1)
    out = pl.pallas_call(
        _gate_body,
        grid=(nblk,),
        in_specs=[
            pl.BlockSpec(qt.shape, lambda i: (0, 0)),
            pl.BlockSpec((_UID_BLK, gate_W.shape[1]), lambda i: (i, 0)),
            pl.BlockSpec((1, 1, _UID_BLK), lambda i: (i, 0, 0)),
        ],
        out_specs=pl.BlockSpec((1, 1, _UID_BLK), lambda i: (i, 0, 0)),
        out_shape=jax.ShapeDtypeStruct((nblk, 1, _UID_BLK), jnp.float32),
    )(qt, gate_W, b3)
    return out.reshape(_N_UIDS // 128, 128)


def _topk_body(mw_ref, tw_ref, ti_ref):
    vals = mw_ref[...]
    ridx = jax.lax.broadcasted_iota(jnp.int32, vals.shape, 0)
    cidx = jax.lax.broadcasted_iota(jnp.int32, vals.shape, 1)
    flat = ridx * 128 + cidx
    tvals, tidxs = [], []
    for _ in range(_TOPK):
        m = jnp.max(vals)
        i = jnp.min(jnp.where(vals == m, flat, _BIGI))
        tvals.append(m)
        tidxs.append(i)
        vals = jnp.where(flat == i, _NEG, vals)

    lane = jax.lax.broadcasted_iota(jnp.int32, (1, 128), 1)
    tw = jnp.zeros((1, 128), jnp.float32)
    ti = jnp.zeros((1, 128), jnp.int32)
    for r in range(_TOPK):
        tw = jnp.where(lane == r, tvals[r], tw)
        ti = jnp.where(lane == r, tidxs[r], ti)
    tw_ref[...] = tw
    ti_ref[...] = ti


def _topk_stage(mw):
    tw, ti = pl.pallas_call(
        _topk_body,
        out_shape=[
            jax.ShapeDtypeStruct((1, 128), jnp.float32),
            jax.ShapeDtypeStruct((1, 128), jnp.int32),
        ],
    )(mw)
    return tw, ti


def _sc_scatter_part(wid, lane, tw_h, ti_h, ow_h, rs_h, sb_v, sb_i, ow_t, rs_t):
    pltpu.sync_copy(tw_h, sb_v)
    pltpu.sync_copy(ti_h, sb_i)
    v0 = sb_v[pl.ds(0, 16)]
    v1 = sb_v[pl.ds(16, 16)]
    i0 = sb_i[pl.ds(0, 16)]
    i1 = sb_i[pl.ds(16, 16)]
    valid1 = lane < (_TOPK - 16)

    mn = jnp.minimum(jnp.min(v0),
                     jnp.min(jnp.where(valid1, v1, jnp.float32(float("inf")))))
    total = jnp.sum(v0 - mn) + jnp.sum(jnp.where(valid1, v1 - mn, 0.0))
    s0 = (v0 - mn) / total
    s1 = jnp.where(valid1, (v1 - mn) / total, 0.0)
    ubase = wid * _USLC
    ik = [jnp.full((16,), i0[k], jnp.int32) for k in range(16)]
    ik += [jnp.full((16,), i1[k - 16], jnp.int32) for k in range(16, _TOPK)]
    sk = [jnp.full((16,), s0[k], jnp.float32) for k in range(16)]
    sk += [jnp.full((16,), s1[k - 16], jnp.float32) for k in range(16, _TOPK)]
    bsz = jnp.full((16,), float(_BATCH), jnp.float32)
    zero16 = jnp.zeros((16,), jnp.float32)
    for j in range(_USLC // 16):
        posg = lane + (ubase + j * 16)
        acc_ow = zero16
        acc_rs = zero16
        for k in range(_TOPK):
            hit = posg == ik[k]
            acc_ow = jnp.where(hit, sk[k], acc_ow)
            acc_rs = jnp.where(hit, bsz, acc_rs)
        ow_t[pl.ds(j * 16, 16)] = acc_ow
        rs_t[pl.ds(j * 16, 16)] = acc_rs
    pltpu.sync_copy(ow_t, ow_h.at[pl.ds(ubase, _USLC)])
    pltpu.sync_copy(rs_t, rs_h.at[pl.ds(ubase, _USLC)])
    return v0, v1


def _sc_body_scatter(tw_h, ti_h, ow_h, rs_h, sb_v, sb_i, ow_t, rs_t):
    cid = lax.axis_index("c")
    sid = lax.axis_index("s")
    wid = sid * 2 + cid
    lane = lax.iota(jnp.int32, 16)
    _sc_scatter_part(wid, lane, tw_h, ti_h, ow_h, rs_h,
                     sb_v, sb_i, ow_t, rs_t)


def _sc_body_join(tw_h, ti_h, resp_h, ow_h, rs_h, wj_h,
                  sb_v, sb_i, ow_t, rs_t, buf, out_t, sem):
    cid = lax.axis_index("c")
    sid = lax.axis_index("s")
    wid = sid * 2 + cid
    lane = lax.iota(jnp.int32, 16)
    v0, v1 = _sc_scatter_part(wid, lane, tw_h, ti_h, ow_h, rs_h,
                              sb_v, sb_i, ow_t, rs_t)

    wk = [jnp.full((16,), v0[k], jnp.float32) for k in range(16)]
    wk += [jnp.full((16,), v1[k - 16], jnp.float32) for k in range(16, _TOPK)]
    rowbase = wid * _NRT
    for c in range(_NRT // _CH if _NRT else 0):
        r0 = rowbase + c * _CH
        descs = [
            pltpu.async_copy(resp_h.at[k, pl.ds(_R_TC + r0, _CH), :],
                             buf.at[k], sem)
            for k in range(_TOPK)
        ]
        for d in descs:
            d.wait()
        for i in range(_CH):
            def jbody(j, carry, i=i):
                acc = buf[0, i, pl.ds(j * 16, 16)] * wk[0]
                for k in range(1, _TOPK):
                    acc = acc + buf[k, i, pl.ds(j * 16, 16)] * wk[k]
                out_t[i, pl.ds(j * 16, 16)] = acc
                return carry
            lax.fori_loop(0, _D // 16, jbody, jnp.int32(0))
        pltpu.sync_copy(out_t, wj_h.at[pl.ds(r0, _CH), :])


def _sc_stage(tw, ti, responses3):
    f32 = jnp.float32
    i32 = jnp.int32
    mesh = plsc.VectorSubcoreMesh(core_axis_name="c", subcore_axis_name="s")
    params = pltpu.CompilerParams(needs_layout_passes=False)
    base_out = [
        jax.ShapeDtypeStruct((_N_UIDS,), f32),
        jax.ShapeDtypeStruct((_N_UIDS,), f32),
    ]
    base_scratch = [
        pltpu.VMEM((32,), f32),
        pltpu.VMEM((32,), i32),
        pltpu.VMEM((_USLC,), f32),
        pltpu.VMEM((_USLC,), f32),
    ]
    if _NJ:
        run = pl.kernel(
            _sc_body_join, mesh=mesh, compiler_params=params,
            out_type=base_out + [jax.ShapeDtypeStruct((_NJ, _D), f32)],
            scratch_types=base_scratch + [
                pltpu.VMEM((_TOPK, _CH, _D), f32),
                pltpu.VMEM((_CH, _D), f32),
                pltpu.SemaphoreType.DMA,
            ],
        )
        return run(tw[0, :32], ti[0, :32], responses3)
    run = pl.kernel(
        _sc_body_scatter, mesh=mesh, compiler_params=params,
        out_type=base_out, scratch_types=base_scratch,
    )
    ow, rs = run(tw[0, :32], ti[0, :32])
    return ow, rs, None


def _join_body(w_ref, r_ref, o_ref):
    acc = r_ref[0] * w_ref[0]
    for i in range(1, _TOPK):
        acc = acc + r_ref[i] * w_ref[i]
    o_ref[...] = acc


def _join_stage(tw20, responses3):
    k = _TOPK
    out = pl.pallas_call(
        _join_body,
        grid=(_R_TC // _ROW_BLK,),
        in_specs=[
            pl.BlockSpec(memory_space=pltpu.SMEM),
            pl.BlockSpec((k, _ROW_BLK, _D), lambda i: (0, i, 0)),
        ],
        out_specs=pl.BlockSpec((_ROW_BLK, _D), lambda i: (i, 0)),
        out_shape=jax.ShapeDtypeStruct((_R_TC, _D), jnp.float32),
    )(tw20, responses3)
    return out


def kernel(query, responses, gate_W, gate_b):
    responses3 = responses.reshape(_TOPK, _ROWS, _D)
    mw = _gate_stage(query, gate_W, gate_b)
    tw, ti = _topk_stage(mw)
    ow, rs, wj_sc = _sc_stage(tw, ti, responses3)
    wj_tc = _join_stage(tw[0, :_TOPK], responses3)
    if _NJ:
        weighted = jnp.concatenate([wj_tc, wj_sc], axis=0)
    else:
        weighted = wj_tc
    return weighted.reshape(_BATCH, _ROWS // _BATCH, _D), ow, rs


# speedup vs baseline: 1.3106x; 1.3106x over previous
import jax
import jax.numpy as jnp
from jax.experimental import pallas as pl
from jax.experimental.pallas import tpu as pltpu

_N_UIDS = 8192
_TOPK = 20
_BATCH = 32
_ROWS = 32 * 128
_D = 512
_QD = 2048
_UID_BLK = 1024
_ROW_BLK = 128
_NG = _N_UIDS // _UID_BLK
_NR = _ROWS // _ROW_BLK
_NS = 4
_PART = _UID_BLK // _NS

_NEG = float("-inf")
_BIGI = 2 ** 30


def _topk_from(mw, tw_ref, ow_ref, rs_ref, wsm_ref):
    ridx = jax.lax.broadcasted_iota(jnp.int32, mw.shape, 0)
    cidx = jax.lax.broadcasted_iota(jnp.int32, mw.shape, 1)
    flat = ridx * 128 + cidx
    vals = mw
    tvals, tidxs = [], []
    for _ in range(_TOPK):
        m = jnp.max(vals)
        i = jnp.min(jnp.where(vals == m, flat, _BIGI))
        tvals.append(m)
        tidxs.append(i)
        vals = jnp.where(flat == i, _NEG, vals)

    lane = jax.lax.broadcasted_iota(jnp.int32, (1, 128), 1)
    tw = jnp.zeros((1, 128), jnp.float32)
    for r in range(_TOPK):
        tw = jnp.where(lane == r, tvals[r], tw)
        wsm_ref[r] = tvals[r]
    tw_ref[...] = tw

    mn = tvals[-1]
    total = tvals[0] - mn
    for r in range(1, _TOPK):
        total = total + (tvals[r] - mn)
    ow = jnp.zeros(mw.shape, jnp.float32)
    member = jnp.zeros(mw.shape, jnp.bool_)
    for r in range(_TOPK):
        hit = flat == tidxs[r]
        ow = jnp.where(hit, (tvals[r] - mn) / total, ow)
        member = jnp.logical_or(member, hit)
    ow_ref[...] = ow
    rs_ref[...] = jnp.where(member, jnp.float32(float(_BATCH)),
                            jnp.float32(0.0))


def _fused_body(qt_ref, w0, w1, w2, w3, b_ref, r_ref,
                o_ref, tw_ref, ow_ref, rs_ref, mw_scr, wsm_ref):
    i = pl.program_id(0)

    @pl.when(i < _NG)
    def _gate():
        dn = (((1,), (0,)), ((), ()))
        for p, w_ref in enumerate((w0, w1, w2, w3)):
            s = jax.lax.dot_general(w_ref[0], qt_ref[...], dn,
                                    preferred_element_type=jnp.float32)
            mv = jnp.sum(s, axis=1) * (1.0 / _BATCH)
            lo = p * _PART
            row = (mv + b_ref[0, 0, lo:lo + _PART]).reshape(_PART // 128, 128)
            mw_scr[pl.ds((i * _UID_BLK + lo) // 128, _PART // 128), :] = row

    @pl.when(i == _NG - 1)
    def _topk():
        _topk_from(mw_scr[...], tw_ref, ow_ref, rs_ref, wsm_ref)

    @pl.when(i >= _NG)
    def _join():
        acc = r_ref[0] * wsm_ref[0]
        for k in range(1, _TOPK):
            acc = acc + r_ref[k] * wsm_ref[k]
        o_ref[...] = acc


def kernel(query, responses, gate_W, gate_b):
    responses3 = responses.reshape(_TOPK, _ROWS, _D)
    qt = jnp.swapaxes(query, 0, 1)
    w4 = gate_W.reshape(_N_UIDS // _PART, _PART, _QD)
    b3 = gate_b.reshape(_NG, 1, _UID_BLK)

    gmax = _NG - 1
    w_specs = [
        pl.BlockSpec(
            (1, _PART, _QD),
            (lambda i, p=p: (_NS * jnp.minimum(i, gmax) + p, 0, 0)))
        for p in range(_NS)
    ]
    weighted, tw, ow, rs = pl.pallas_call(
        _fused_body,
        grid=(_NG + _NR,),
        in_specs=[pl.BlockSpec(qt.shape, lambda i: (0, 0))] + w_specs + [
            pl.BlockSpec((1, 1, _UID_BLK),
                         lambda i: (jnp.minimum(i, gmax), 0, 0)),
            pl.BlockSpec((_TOPK, _ROW_BLK, _D),
                         lambda i: (0, jnp.maximum(i - _NG, 0), 0)),
        ],
        out_specs=[
            pl.BlockSpec((_ROW_BLK, _D), lambda i: (jnp.maximum(i - _NG, 0), 0)),
            pl.BlockSpec((1, 128), lambda i: (0, 0)),
            pl.BlockSpec((_N_UIDS // 128, 128), lambda i: (0, 0)),
            pl.BlockSpec((_N_UIDS // 128, 128), lambda i: (0, 0)),
        ],
        out_shape=[
            jax.ShapeDtypeStruct((_ROWS, _D), jnp.float32),
            jax.ShapeDtypeStruct((1, 128), jnp.float32),
            jax.ShapeDtypeStruct((_N_UIDS // 128, 128), jnp.float32),
            jax.ShapeDtypeStruct((_N_UIDS // 128, 128), jnp.float32),
        ],
        scratch_shapes=[
            pltpu.VMEM((_N_UIDS // 128, 128), jnp.float32),
            pltpu.SMEM((_TOPK,), jnp.float32),
        ],
    )(qt, w4, w4, w4, w4, b3, responses3)
    del tw
    return (weighted.reshape(_BATCH, _ROWS // _BATCH, _D),
            ow.reshape(_N_UIDS), rs.reshape(_N_UIDS))


# speedup vs baseline: 1.3159x; 1.0040x over previous
import jax
import jax.numpy as jnp
from jax.experimental import pallas as pl
from jax.experimental.pallas import tpu as pltpu

_N_UIDS = 8192
_TOPK = 20
_BATCH = 32
_ROWS = 32 * 128
_D = 512
_QD = 2048
_UID_BLK = 1024
_ROW_BLK = 256
_NG = _N_UIDS // _UID_BLK
_NR = _ROWS // _ROW_BLK
_NS = 4
_PART = _UID_BLK // _NS

_NEG = float("-inf")
_BIGI = 2 ** 30


def _topk_from(mw, tw_ref, ow_ref, rs_ref, wsm_ref):
    ridx = jax.lax.broadcasted_iota(jnp.int32, mw.shape, 0)
    cidx = jax.lax.broadcasted_iota(jnp.int32, mw.shape, 1)
    flat = ridx * 128 + cidx
    vals = mw
    tvals, tidxs = [], []
    for _ in range(_TOPK):
        m = jnp.max(vals)
        i = jnp.min(jnp.where(vals == m, flat, _BIGI))
        tvals.append(m)
        tidxs.append(i)
        vals = jnp.where(flat == i, _NEG, vals)

    lane = jax.lax.broadcasted_iota(jnp.int32, (1, 128), 1)
    tw = jnp.zeros((1, 128), jnp.float32)
    for r in range(_TOPK):
        tw = jnp.where(lane == r, tvals[r], tw)
        wsm_ref[r] = tvals[r]
    tw_ref[...] = tw

    mn = tvals[-1]
    total = tvals[0] - mn
    for r in range(1, _TOPK):
        total = total + (tvals[r] - mn)
    ow = jnp.zeros(mw.shape, jnp.float32)
    member = jnp.zeros(mw.shape, jnp.bool_)
    for r in range(_TOPK):
        hit = flat == tidxs[r]
        ow = jnp.where(hit, (tvals[r] - mn) / total, ow)
        member = jnp.logical_or(member, hit)
    ow_ref[...] = ow
    rs_ref[...] = jnp.where(member, jnp.float32(float(_BATCH)),
                            jnp.float32(0.0))


def _fused_body(qt_ref, w0, w1, w2, w3, b_ref, r_ref,
                o_ref, tw_ref, ow_ref, rs_ref, mw_scr, wsm_ref):
    i = pl.program_id(0)

    @pl.when(i < _NG)
    def _gate():
        dn = (((1,), (0,)), ((), ()))
        for p, w_ref in enumerate((w0, w1, w2, w3)):
            s = jax.lax.dot_general(w_ref[0], qt_ref[...], dn,
                                    preferred_element_type=jnp.float32)
            mv = jnp.sum(s, axis=1) * (1.0 / _BATCH)
            lo = p * _PART
            row = (mv + b_ref[0, 0, lo:lo + _PART]).reshape(_PART // 128, 128)
            mw_scr[pl.ds((i * _UID_BLK + lo) // 128, _PART // 128), :] = row

    @pl.when(i == _NG - 1)
    def _topk():
        _topk_from(mw_scr[...], tw_ref, ow_ref, rs_ref, wsm_ref)

    @pl.when(i >= _NG)
    def _join():
        acc = r_ref[0] * wsm_ref[0]
        for k in range(1, _TOPK):
            acc = acc + r_ref[k] * wsm_ref[k]
        o_ref[...] = acc


def kernel(query, responses, gate_W, gate_b):
    responses3 = responses.reshape(_TOPK, _ROWS, _D)
    qt = jnp.swapaxes(query, 0, 1)
    w4 = gate_W.reshape(_N_UIDS // _PART, _PART, _QD)
    b3 = gate_b.reshape(_NG, 1, _UID_BLK)

    gmax = _NG - 1
    w_specs = [
        pl.BlockSpec(
            (1, _PART, _QD),
            (lambda i, p=p: (_NS * jnp.minimum(i, gmax) + p, 0, 0)))
        for p in range(_NS)
    ]
    weighted, tw, ow, rs = pl.pallas_call(
        _fused_body,
        grid=(_NG + _NR,),
        in_specs=[pl.BlockSpec(qt.shape, lambda i: (0, 0))] + w_specs + [
            pl.BlockSpec((1, 1, _UID_BLK),
                         lambda i: (jnp.minimum(i, gmax), 0, 0)),
            pl.BlockSpec((_TOPK, _ROW_BLK, _D),
                         lambda i: (0, jnp.maximum(i - _NG, 0), 0)),
        ],
        out_specs=[
            pl.BlockSpec((_ROW_BLK, _D), lambda i: (jnp.maximum(i - _NG, 0), 0)),
            pl.BlockSpec((1, 128), lambda i: (0, 0)),
            pl.BlockSpec((_N_UIDS // 128, 128), lambda i: (0, 0)),
            pl.BlockSpec((_N_UIDS // 128, 128), lambda i: (0, 0)),
        ],
        out_shape=[
            jax.ShapeDtypeStruct((_ROWS, _D), jnp.float32),
            jax.ShapeDtypeStruct((1, 128), jnp.float32),
            jax.ShapeDtypeStruct((_N_UIDS // 128, 128), jnp.float32),
            jax.ShapeDtypeStruct((_N_UIDS // 128, 128), jnp.float32),
        ],
        scratch_shapes=[
            pltpu.VMEM((_N_UIDS // 128, 128), jnp.float32),
            pltpu.SMEM((_TOPK,), jnp.float32),
        ],
    )(qt, w4, w4, w4, w4, b3, responses3)
    del tw
    return (weighted.reshape(_BATCH, _ROWS // _BATCH, _D),
            ow.reshape(_N_UIDS), rs.reshape(_N_UIDS))
